# baseline (speedup 1.0000x reference)
import functools

import jax
import jax.numpy as jnp
from jax import lax
from jax.experimental import pallas as pl
from jax.experimental.pallas import tpu as pltpu
from jax.experimental.pallas import tpu_sc as plsc

N = 10000
NPAD = 10240
E = 160000
H = 128
K = 3
R = 4

NHALF = N // 2
ACC_ROWS = 5120
DUMP_ROW = 5008
CH = 80
TILE_E = E // 16
NCH = TILE_E // CH
SLICE = NPAD // 16


def _sc_body(s_hbm, rel_hbm, pk_hbm, p_out, n_out,
             wloc, red16,
             pkb0, idxb0, mpb0,
             pkb1, idxb1, mpb1,
             mnb, acc_p, acc_n, lsem0, lsem1, ssem0, ssem1):
    c = lax.axis_index("c")
    sid = lax.axis_index("s")
    base_node = c * NHALF
    ebase = sid * TILE_E

    bufs = ((pkb0, idxb0, mpb0, lsem0),
            (pkb1, idxb1, mpb1, lsem1))

    def start_load(ci, b):
        pkb, idxb, mpb, lsem = bufs[b]
        off = ebase + ci * CH
        pltpu.async_copy(pk_hbm.at[pl.ds(off, CH)], pkb, lsem)
        pltpu.async_copy(rel_hbm.at[pl.ds(off, CH)], mpb, lsem)

    def wait_load(b):
        pkb, idxb, mpb, lsem = bufs[b]
        pltpu.make_async_copy(pk_hbm.at[pl.ds(0, CH)], pkb, lsem).wait()
        pltpu.make_async_copy(rel_hbm.at[pl.ds(0, CH)], mpb, lsem).wait()

    def start_scat(b):
        pkb, idxb, mpb, lsem = bufs[b]
        pltpu.async_copy(mpb, acc_p.at[idxb], ssem0, add=True)
        pltpu.async_copy(mnb, acc_n.at[idxb], ssem1, add=True)

    def wait_scat(b):
        pkb, idxb, mpb, lsem = bufs[b]
        pltpu.make_async_copy(mpb, acc_p.at[idxb], ssem0).wait()
        pltpu.make_async_copy(mnb, acc_n.at[idxb], ssem1).wait()

    start_load(0, 0)

    zero16 = jnp.zeros((16,), jnp.float32)

    def _zrow(i, carry):
        for j in range(8):
            mnb[i, pl.ds(16 * j, 16)] = zero16
        return carry

    lax.fori_loop(0, CH, _zrow, 0)
    for kc in range(4):
        pltpu.sync_copy(mnb, acc_p.at[pl.ds(sid * 320 + kc * CH, CH)])
        pltpu.sync_copy(mnb, acc_n.at[pl.ds(sid * 320 + kc * CH, CH)])

    plsc.subcore_barrier()

    pltpu.sync_copy(s_hbm, wloc)

    def _maxb(g, m):
        return jnp.maximum(m, wloc[pl.ds(16 * g, 16)])

    mv = lax.fori_loop(1, NPAD // 16, _maxb, wloc[pl.ds(0, 16)])
    red16[pl.ds(0, 16)] = mv
    gmax = plsc.load_gather(red16, [jnp.zeros((16,), jnp.int32)])
    for j in range(1, 16):
        gmax = jnp.maximum(
            gmax, plsc.load_gather(red16, [jnp.full((16,), j, jnp.int32)]))

    def _expb(g, a):
        ev = jnp.exp(wloc[pl.ds(16 * g, 16)] - gmax)
        wloc[pl.ds(16 * g, 16)] = ev
        return a + ev

    zacc = lax.fori_loop(0, NPAD // 16, _expb, jnp.zeros((16,), jnp.float32))
    red16[pl.ds(0, 16)] = zacc
    zsum = plsc.load_gather(red16, [jnp.zeros((16,), jnp.int32)])
    for j in range(1, 16):
        zsum = zsum + plsc.load_gather(red16, [jnp.full((16,), j, jnp.int32)])
    inv_z = 1.0 / zsum

    def compute(b):
        pkb, idxb, mpb, lsem = bufs[b]

        def _group(g, gc):
            pk = pkb[pl.ds(16 * g, 16)]
            sv = pk & 0xFFFF
            dv = lax.shift_right_logical(pk, 16)
            owned = (dv >= base_node) & (dv < base_node + NHALF)
            idxb[pl.ds(16 * g, 16)] = jnp.where(owned, dv - base_node,
                                                DUMP_ROW)
            wv = plsc.load_gather(wloc, [sv]) * inv_z
            for je in range(16):
                sp = wv[je]
                row = 16 * g + je
                for j in range(8):
                    rv = mpb[row, pl.ds(16 * j, 16)]
                    pv = jnp.maximum(rv, 0.0)
                    mpb[row, pl.ds(16 * j, 16)] = pv * sp
                    mnb[row, pl.ds(16 * j, 16)] = (pv - rv) * sp
            return gc

        lax.fori_loop(0, CH // 16, _group, 0, unroll=5)

    def step(ci, b):
        @pl.when(ci >= 1)
        def _():
            wait_scat(1 - b)

        @pl.when(ci < NCH - 1)
        def _():
            start_load(ci + 1, 1 - b)

        wait_load(b)
        compute(b)
        start_scat(b)

    def _loop(ci, carry):
        @pl.when(ci % 2 == 0)
        def _():
            step(ci, 0)

        @pl.when(ci % 2 == 1)
        def _():
            step(ci, 1)

        return carry

    lax.fori_loop(0, NCH, _loop, 0)
    wait_scat((NCH - 1) % 2)
    plsc.subcore_barrier()

    r0 = sid * 320
    pltpu.sync_copy(acc_p.at[pl.ds(r0, 320)], p_out.at[c, pl.ds(r0, 320)])
    pltpu.sync_copy(acc_n.at[pl.ds(r0, 320)], n_out.at[c, pl.ds(r0, 320)])


_sc_step = functools.partial(
    pl.kernel,
    out_type=(
        jax.ShapeDtypeStruct((2, ACC_ROWS, H), jnp.float32),
        jax.ShapeDtypeStruct((2, ACC_ROWS, H), jnp.float32),
    ),
    mesh=plsc.VectorSubcoreMesh(core_axis_name="c", subcore_axis_name="s"),
    compiler_params=pltpu.CompilerParams(needs_layout_passes=False),
    scratch_types=[
        pltpu.VMEM((NPAD,), jnp.float32),
        pltpu.VMEM((128,), jnp.float32),
        pltpu.VMEM((CH,), jnp.int32),
        pltpu.VMEM((CH,), jnp.int32),
        pltpu.VMEM((CH, H), jnp.float32),
        pltpu.VMEM((CH,), jnp.int32),
        pltpu.VMEM((CH,), jnp.int32),
        pltpu.VMEM((CH, H), jnp.float32),
        pltpu.VMEM((CH, H), jnp.float32),
        pltpu.VMEM_SHARED((ACC_ROWS, H), jnp.float32),
        pltpu.VMEM_SHARED((ACC_ROWS, H), jnp.float32),
        pltpu.SemaphoreType.DMA,
        pltpu.SemaphoreType.DMA,
        pltpu.SemaphoreType.DMA,
        pltpu.SemaphoreType.DMA,
    ],
)(_sc_body)


def _dense_body(h_ref, h0_ref, p_ref, ng_ref, wht_ref, mp_ref, mn_ref,
                bf1_ref, wf2t_ref, bf2_ref, wiht_ref, bih_ref, whht_ref,
                bhh_ref, lng_ref, lnb_ref, wot_ref, bo_ref, hn_ref, s_ref):
    f32 = jnp.float32
    hb = h_ref[...]
    u = jnp.dot(hb, wht_ref[...], preferred_element_type=f32)
    u = u + jnp.dot(p_ref[0], mp_ref[...], preferred_element_type=f32)
    u = u + jnp.dot(ng_ref[0], mn_ref[...], preferred_element_type=f32)
    u = u + bf1_ref[...]
    g = u * 0.5 * (1.0 + lax.erf(u * 0.7071067811865476))
    a = jnp.dot(g, wf2t_ref[...], preferred_element_type=f32)
    a = a + bf2_ref[...] + 0.1 * h0_ref[...]
    gi = jnp.dot(a, wiht_ref[...], preferred_element_type=f32) + bih_ref[...]
    gh = jnp.dot(hb, whht_ref[...], preferred_element_type=f32) + bhh_ref[...]
    r = jax.nn.sigmoid(gi[:, 0:H] + gh[:, 0:H])
    z = jax.nn.sigmoid(gi[:, H:2 * H] + gh[:, H:2 * H])
    ng = jnp.tanh(gi[:, 2 * H:3 * H] + r * gh[:, 2 * H:3 * H])
    hn = (1.0 - z) * ng + z * hb
    mu = jnp.mean(hn, axis=-1, keepdims=True)
    var = jnp.mean((hn - mu) ** 2, axis=-1, keepdims=True)
    hn = (hn - mu) * lax.rsqrt(var + 1e-5) * lng_ref[...] + lnb_ref[...]
    hn_ref[...] = hn
    s_ref[...] = jnp.dot(hn, wot_ref[...], preferred_element_type=f32) + bo_ref[...]


_BLK = 1000


def _dense_call(h, h0, p, ngv, wht, mp, mn, bf1, wf2t, bf2, wiht, bih,
                whht, bhh, lng, lnb, wot, bo):
    full = lambda shape: pl.BlockSpec(shape, lambda i: (0,) * len(shape))
    return pl.pallas_call(
        _dense_body,
        grid=(N // _BLK,),
        in_specs=[
            pl.BlockSpec((_BLK, H), lambda i: (i, 0)),
            pl.BlockSpec((_BLK, H), lambda i: (i, 0)),
            pl.BlockSpec((1, _BLK, H), lambda i: (i // 5, i % 5, 0)),
            pl.BlockSpec((1, _BLK, H), lambda i: (i // 5, i % 5, 0)),
            full((H, H)), full((H, H)), full((H, H)),
            full((1, H)), full((H, H)), full((1, H)),
            full((H, 3 * H)), full((1, 3 * H)), full((H, 3 * H)),
            full((1, 3 * H)), full((1, H)), full((1, H)),
            full((H, 1)), full((1, 1)),
        ],
        out_specs=[
            pl.BlockSpec((_BLK, H), lambda i: (i, 0)),
            pl.BlockSpec((_BLK, 1), lambda i: (i, 0)),
        ],
        out_shape=[
            jax.ShapeDtypeStruct((N, H), jnp.float32),
            jax.ShapeDtypeStruct((N, 1), jnp.float32),
        ],
    )(h, h0, p, ngv, wht, mp, mn, bf1, wf2t, bf2, wiht, bih, whht, bhh,
      lng, lnb, wot, bo)


def _matmul_bias_body(x_ref, w_ref, b_ref, o_ref):
    o_ref[...] = jnp.dot(x_ref[...], w_ref[...],
                         preferred_element_type=jnp.float32) + b_ref[...]


def _matmul_bias(x, w, b, blk):
    rows = x.shape[0]
    return pl.pallas_call(
        _matmul_bias_body,
        grid=(rows // blk,),
        in_specs=[
            pl.BlockSpec((blk, H), lambda i: (i, 0)),
            pl.BlockSpec((H, H), lambda i: (0, 0)),
            pl.BlockSpec((1, H), lambda i: (0, 0)),
        ],
        out_specs=pl.BlockSpec((blk, H), lambda i: (i, 0)),
        out_shape=jax.ShapeDtypeStruct((rows, H), jnp.float32),
    )(x, w, b)


def kernel(node_emb, node_mask, seed_mask, edge_src, edge_dst, edge_rel_emb,
           edge_dir, edge_mask, q_emb, Wn, bn, Wr, br, Wq, bq, Wins, bins,
           Wf1, bf1, Wf2, bf2, Wih, bih, Whh, bhh, ln_g, ln_b, Wsc, bsc,
           Wout, bout):
    f32 = jnp.float32
    src = edge_src[0].astype(jnp.int32)
    dst = edge_dst[0].astype(jnp.int32)
    pk = src | (dst << 16)

    qh = (q_emb @ Wq.T + bq)[0]
    ins = (qh @ Wins.T + bins).reshape(K, H)
    rk = jax.nn.relu(ins)
    sk = jax.nn.relu(-ins)
    mp = jnp.zeros((H, H), f32)
    mn = jnp.zeros((H, H), f32)
    for k in range(K):
        wk_t = Wf1[:, (1 + k) * H:(2 + k) * H].T
        mp = mp + rk[k][:, None] * wk_t
        mn = mn + sk[k][:, None] * wk_t
    wht = Wf1[:, :H].T

    h = _matmul_bias(node_emb[0], Wn.T, (bn + qh).reshape(1, H), _BLK)
    rel = _matmul_bias(edge_rel_emb[0], Wr.T, br.reshape(1, H), 2000)
    h0 = h

    pad = jnp.full((NPAD - N,), -1e30, f32)
    s = jnp.concatenate(
        [jnp.where(seed_mask[0], 0.0, -1e4).astype(f32), pad])

    wf2t = Wf2.T
    wiht = Wih.T
    whht = Whh.T
    bf1r = bf1.reshape(1, H)
    bf2r = bf2.reshape(1, H)
    bihr = bih.reshape(1, 3 * H)
    bhhr = bhh.reshape(1, 3 * H)
    lngr = ln_g.reshape(1, H)
    lnbr = ln_b.reshape(1, H)
    wsct = Wsc.T
    bscr = bsc.reshape(1, 1)
    woutt = Wout.T
    boutr = bout.reshape(1, 1)

    for step in range(R):
        p_acc, n_acc = _sc_step(s, rel, pk)
        last = step == R - 1
        h, s2d = _dense_call(
            h, h0, p_acc, n_acc, wht, mp, mn, bf1r, wf2t, bf2r, wiht, bihr,
            whht, bhhr, lngr, lnbr,
            woutt if last else wsct, boutr if last else bscr)
        if not last:
            s = jnp.concatenate([s2d[:, 0], pad])

    return s2d[:, 0][None, :]


# speedup vs baseline: 1.2042x; 1.2042x over previous
import functools

import jax
import jax.numpy as jnp
from jax import lax
from jax.experimental import pallas as pl
from jax.experimental.pallas import tpu as pltpu
from jax.experimental.pallas import tpu_sc as plsc

N = 10000
NPAD = 10240
E = 160000
H = 128
K = 3
R = 4

NHALF = N // 2
ACC_ROWS = 5120
DUMP_ROW = 5008
CH = 80
TILE_E = E // 16
NCH = TILE_E // CH
SLICE = NPAD // 16


def _sc_body(s_hbm, rel_hbm, pk_hbm, p_out, n_out,
             wloc, red16,
             pkb0, idxb0, mpb0,
             pkb1, idxb1, mpb1,
             mnb, acc_p, acc_n, lsem0, lsem1, ssem0, ssem1):
    c = lax.axis_index("c")
    sid = lax.axis_index("s")
    base_node = c * NHALF
    ebase = sid * TILE_E

    bufs = ((pkb0, idxb0, mpb0, lsem0),
            (pkb1, idxb1, mpb1, lsem1))

    def start_load(ci, b):
        pkb, idxb, mpb, lsem = bufs[b]
        off = ebase + ci * CH
        pltpu.async_copy(pk_hbm.at[pl.ds(off, CH)], pkb, lsem)
        pltpu.async_copy(rel_hbm.at[pl.ds(off, CH)], mpb, lsem)

    def wait_load(b):
        pkb, idxb, mpb, lsem = bufs[b]
        pltpu.make_async_copy(pk_hbm.at[pl.ds(0, CH)], pkb, lsem).wait()
        pltpu.make_async_copy(rel_hbm.at[pl.ds(0, CH)], mpb, lsem).wait()

    def start_scat(b):
        pkb, idxb, mpb, lsem = bufs[b]
        pltpu.async_copy(mpb, acc_p.at[idxb], ssem0, add=True)
        pltpu.async_copy(mnb, acc_n.at[idxb], ssem1, add=True)

    def wait_scat(b):
        pkb, idxb, mpb, lsem = bufs[b]
        pltpu.make_async_copy(mpb, acc_p.at[idxb], ssem0).wait()
        pltpu.make_async_copy(mnb, acc_n.at[idxb], ssem1).wait()

    start_load(0, 0)

    zero16 = jnp.zeros((16,), jnp.float32)

    def _zrow(i, carry):
        for j in range(8):
            mnb[i, pl.ds(16 * j, 16)] = zero16
        return carry

    lax.fori_loop(0, CH, _zrow, 0)
    for kc in range(4):
        pltpu.sync_copy(mnb, acc_p.at[pl.ds(sid * 320 + kc * CH, CH)])
        pltpu.sync_copy(mnb, acc_n.at[pl.ds(sid * 320 + kc * CH, CH)])

    plsc.subcore_barrier()

    pltpu.sync_copy(s_hbm, wloc)

    def _maxb(g, m):
        return jnp.maximum(m, wloc[pl.ds(16 * g, 16)])

    mv = lax.fori_loop(1, NPAD // 16, _maxb, wloc[pl.ds(0, 16)])
    red16[pl.ds(0, 16)] = mv
    gmax = plsc.load_gather(red16, [jnp.zeros((16,), jnp.int32)])
    for j in range(1, 16):
        gmax = jnp.maximum(
            gmax, plsc.load_gather(red16, [jnp.full((16,), j, jnp.int32)]))

    def _expb(g, a):
        ev = jnp.exp(wloc[pl.ds(16 * g, 16)] - gmax)
        wloc[pl.ds(16 * g, 16)] = ev
        return a + ev

    zacc = lax.fori_loop(0, NPAD // 16, _expb, jnp.zeros((16,), jnp.float32))
    red16[pl.ds(0, 16)] = zacc
    zsum = plsc.load_gather(red16, [jnp.zeros((16,), jnp.int32)])
    for j in range(1, 16):
        zsum = zsum + plsc.load_gather(red16, [jnp.full((16,), j, jnp.int32)])
    inv_z = 1.0 / zsum

    def compute(b):
        pkb, idxb, mpb, lsem = bufs[b]

        def _group(g, gc):
            pk = pkb[pl.ds(16 * g, 16)]
            sv = pk & 0xFFFF
            dv = lax.shift_right_logical(pk, 16)
            owned = (dv >= base_node) & (dv < base_node + NHALF)
            idxb[pl.ds(16 * g, 16)] = jnp.where(owned, dv - base_node,
                                                DUMP_ROW)
            wv = plsc.load_gather(wloc, [sv]) * inv_z
            for je in range(16):
                sp = wv[je]
                row = 16 * g + je
                for j in range(8):
                    rv = mpb[row, pl.ds(16 * j, 16)]
                    pv = jnp.maximum(rv, 0.0)
                    mpb[row, pl.ds(16 * j, 16)] = pv * sp
                    mnb[row, pl.ds(16 * j, 16)] = (pv - rv) * sp
            return gc

        lax.fori_loop(0, CH // 16, _group, 0)

    def step(ci, b):
        @pl.when(ci >= 1)
        def _():
            wait_scat(1 - b)

        @pl.when(ci < NCH - 1)
        def _():
            start_load(ci + 1, 1 - b)

        wait_load(b)
        compute(b)
        start_scat(b)

    def _loop(ci, carry):
        @pl.when(ci % 2 == 0)
        def _():
            step(ci, 0)

        @pl.when(ci % 2 == 1)
        def _():
            step(ci, 1)

        return carry

    lax.fori_loop(0, NCH, _loop, 0)
    wait_scat((NCH - 1) % 2)
    plsc.subcore_barrier()

    r0 = sid * 320
    pltpu.sync_copy(acc_p.at[pl.ds(r0, 320)], p_out.at[c, pl.ds(r0, 320)])
    pltpu.sync_copy(acc_n.at[pl.ds(r0, 320)], n_out.at[c, pl.ds(r0, 320)])


_sc_step = functools.partial(
    pl.kernel,
    out_type=(
        jax.ShapeDtypeStruct((2, ACC_ROWS, H), jnp.float32),
        jax.ShapeDtypeStruct((2, ACC_ROWS, H), jnp.float32),
    ),
    mesh=plsc.VectorSubcoreMesh(core_axis_name="c", subcore_axis_name="s"),
    compiler_params=pltpu.CompilerParams(needs_layout_passes=False),
    scratch_types=[
        pltpu.VMEM((NPAD,), jnp.float32),
        pltpu.VMEM((128,), jnp.float32),
        pltpu.VMEM((CH,), jnp.int32),
        pltpu.VMEM((CH,), jnp.int32),
        pltpu.VMEM((CH, H), jnp.float32),
        pltpu.VMEM((CH,), jnp.int32),
        pltpu.VMEM((CH,), jnp.int32),
        pltpu.VMEM((CH, H), jnp.float32),
        pltpu.VMEM((CH, H), jnp.float32),
        pltpu.VMEM_SHARED((ACC_ROWS, H), jnp.float32),
        pltpu.VMEM_SHARED((ACC_ROWS, H), jnp.float32),
        pltpu.SemaphoreType.DMA,
        pltpu.SemaphoreType.DMA,
        pltpu.SemaphoreType.DMA,
        pltpu.SemaphoreType.DMA,
    ],
)(_sc_body)


def _dense_body(h_ref, h0_ref, p_ref, ng_ref, wht_ref, mp_ref, mn_ref,
                bf1_ref, wf2t_ref, bf2_ref, wiht_ref, bih_ref, whht_ref,
                bhh_ref, lng_ref, lnb_ref, wot_ref, bo_ref, hn_ref, s_ref):
    f32 = jnp.float32
    hb = h_ref[...]
    u = jnp.dot(hb, wht_ref[...], preferred_element_type=f32)
    u = u + jnp.dot(p_ref[0], mp_ref[...], preferred_element_type=f32)
    u = u + jnp.dot(ng_ref[0], mn_ref[...], preferred_element_type=f32)
    u = u + bf1_ref[...]
    g = u * 0.5 * (1.0 + lax.erf(u * 0.7071067811865476))
    a = jnp.dot(g, wf2t_ref[...], preferred_element_type=f32)
    a = a + bf2_ref[...] + 0.1 * h0_ref[...]
    gi = jnp.dot(a, wiht_ref[...], preferred_element_type=f32) + bih_ref[...]
    gh = jnp.dot(hb, whht_ref[...], preferred_element_type=f32) + bhh_ref[...]
    r = jax.nn.sigmoid(gi[:, 0:H] + gh[:, 0:H])
    z = jax.nn.sigmoid(gi[:, H:2 * H] + gh[:, H:2 * H])
    ng = jnp.tanh(gi[:, 2 * H:3 * H] + r * gh[:, 2 * H:3 * H])
    hn = (1.0 - z) * ng + z * hb
    mu = jnp.mean(hn, axis=-1, keepdims=True)
    var = jnp.mean((hn - mu) ** 2, axis=-1, keepdims=True)
    hn = (hn - mu) * lax.rsqrt(var + 1e-5) * lng_ref[...] + lnb_ref[...]
    hn_ref[...] = hn
    s_ref[...] = jnp.dot(hn, wot_ref[...], preferred_element_type=f32) + bo_ref[...]


_BLK = 1000


def _dense_call(h, h0, p, ngv, wht, mp, mn, bf1, wf2t, bf2, wiht, bih,
                whht, bhh, lng, lnb, wot, bo):
    full = lambda shape: pl.BlockSpec(shape, lambda i: (0,) * len(shape))
    return pl.pallas_call(
        _dense_body,
        grid=(N // _BLK,),
        in_specs=[
            pl.BlockSpec((_BLK, H), lambda i: (i, 0)),
            pl.BlockSpec((_BLK, H), lambda i: (i, 0)),
            pl.BlockSpec((1, _BLK, H), lambda i: (i // 5, i % 5, 0)),
            pl.BlockSpec((1, _BLK, H), lambda i: (i // 5, i % 5, 0)),
            full((H, H)), full((H, H)), full((H, H)),
            full((1, H)), full((H, H)), full((1, H)),
            full((H, 3 * H)), full((1, 3 * H)), full((H, 3 * H)),
            full((1, 3 * H)), full((1, H)), full((1, H)),
            full((H, 1)), full((1, 1)),
        ],
        out_specs=[
            pl.BlockSpec((_BLK, H), lambda i: (i, 0)),
            pl.BlockSpec((_BLK, 1), lambda i: (i, 0)),
        ],
        out_shape=[
            jax.ShapeDtypeStruct((N, H), jnp.float32),
            jax.ShapeDtypeStruct((N, 1), jnp.float32),
        ],
    )(h, h0, p, ngv, wht, mp, mn, bf1, wf2t, bf2, wiht, bih, whht, bhh,
      lng, lnb, wot, bo)


def _matmul_bias_body(x_ref, w_ref, b_ref, o_ref):
    o_ref[...] = jnp.dot(x_ref[...], w_ref[...],
                         preferred_element_type=jnp.float32) + b_ref[...]


def _matmul_bias(x, w, b, blk):
    rows = x.shape[0]
    return pl.pallas_call(
        _matmul_bias_body,
        grid=(rows // blk,),
        in_specs=[
            pl.BlockSpec((blk, H), lambda i: (i, 0)),
            pl.BlockSpec((H, H), lambda i: (0, 0)),
            pl.BlockSpec((1, H), lambda i: (0, 0)),
        ],
        out_specs=pl.BlockSpec((blk, H), lambda i: (i, 0)),
        out_shape=jax.ShapeDtypeStruct((rows, H), jnp.float32),
    )(x, w, b)


def kernel(node_emb, node_mask, seed_mask, edge_src, edge_dst, edge_rel_emb,
           edge_dir, edge_mask, q_emb, Wn, bn, Wr, br, Wq, bq, Wins, bins,
           Wf1, bf1, Wf2, bf2, Wih, bih, Whh, bhh, ln_g, ln_b, Wsc, bsc,
           Wout, bout):
    f32 = jnp.float32
    src = edge_src[0].astype(jnp.int32)
    dst = edge_dst[0].astype(jnp.int32)
    pk = src | (dst << 16)

    qh = (q_emb @ Wq.T + bq)[0]
    ins = (qh @ Wins.T + bins).reshape(K, H)
    rk = jax.nn.relu(ins)
    sk = jax.nn.relu(-ins)
    mp = jnp.zeros((H, H), f32)
    mn = jnp.zeros((H, H), f32)
    for k in range(K):
        wk_t = Wf1[:, (1 + k) * H:(2 + k) * H].T
        mp = mp + rk[k][:, None] * wk_t
        mn = mn + sk[k][:, None] * wk_t
    wht = Wf1[:, :H].T

    h = _matmul_bias(node_emb[0], Wn.T, (bn + qh).reshape(1, H), _BLK)
    rel = _matmul_bias(edge_rel_emb[0], Wr.T, br.reshape(1, H), 2000)
    h0 = h

    pad = jnp.full((NPAD - N,), -1e30, f32)
    s = jnp.concatenate(
        [jnp.where(seed_mask[0], 0.0, -1e4).astype(f32), pad])

    wf2t = Wf2.T
    wiht = Wih.T
    whht = Whh.T
    bf1r = bf1.reshape(1, H)
    bf2r = bf2.reshape(1, H)
    bihr = bih.reshape(1, 3 * H)
    bhhr = bhh.reshape(1, 3 * H)
    lngr = ln_g.reshape(1, H)
    lnbr = ln_b.reshape(1, H)
    wsct = Wsc.T
    bscr = bsc.reshape(1, 1)
    woutt = Wout.T
    boutr = bout.reshape(1, 1)

    for step in range(R):
        p_acc, n_acc = _sc_step(s, rel, pk)
        last = step == R - 1
        h, s2d = _dense_call(
            h, h0, p_acc, n_acc, wht, mp, mn, bf1r, wf2t, bf2r, wiht, bihr,
            whht, bhhr, lngr, lnbr,
            woutt if last else wsct, boutr if last else bscr)
        if not last:
            s = jnp.concatenate([s2d[:, 0], pad])

    return s2d[:, 0][None, :]


# speedup vs baseline: 1.3668x; 1.1350x over previous
import functools

import jax
import jax.numpy as jnp
from jax import lax
from jax.experimental import pallas as pl
from jax.experimental.pallas import tpu as pltpu
from jax.experimental.pallas import tpu_sc as plsc

N = 10000
NPAD = 10240
E = 160000
H = 128
K = 3
R = 4

NHALF = N // 2
ACC_ROWS = 5120
DUMP_ROW = 5008
CH = 80
TILE_E = E // 16
NCH = TILE_E // CH
SLICE = NPAD // 16


def _sc_body(s_hbm, rel_hbm, pk_hbm, p_out, n_out,
             wloc, red16,
             pkb0, ixa0, ixb0, mpb0,
             pkb1, ixa1, ixb1, mpb1,
             mnb, acc_p, acc_n, lsem0, lsem1, ssem0, ssem1):
    c = lax.axis_index("c")
    sid = lax.axis_index("s")
    base_node = c * NHALF
    ebase = sid * TILE_E

    bufs = ((pkb0, ixa0, ixb0, mpb0, lsem0),
            (pkb1, ixa1, ixb1, mpb1, lsem1))

    def start_load(ci, b):
        pkb, ixa, ixb, mpb, lsem = bufs[b]
        off = ebase + ci * CH
        pltpu.async_copy(pk_hbm.at[pl.ds(off, CH)], pkb, lsem)
        pltpu.async_copy(rel_hbm.at[pl.ds(off, CH)], mpb, lsem)

    def wait_load(b):
        pkb, ixa, ixb, mpb, lsem = bufs[b]
        pltpu.make_async_copy(pk_hbm.at[pl.ds(0, CH)], pkb, lsem).wait()
        pltpu.make_async_copy(rel_hbm.at[pl.ds(0, CH)], mpb, lsem).wait()

    def start_scat_a(b):
        pkb, ixa, ixb, mpb, lsem = bufs[b]
        pltpu.async_copy(mpb.at[pl.ds(0, 48)], acc_p.at[ixa], ssem0, add=True)
        pltpu.async_copy(mnb.at[pl.ds(0, 48)], acc_n.at[ixa], ssem0, add=True)

    def start_scat_b(b):
        pkb, ixa, ixb, mpb, lsem = bufs[b]
        pltpu.async_copy(mpb.at[pl.ds(48, 32)], acc_p.at[ixb], ssem1, add=True)
        pltpu.async_copy(mnb.at[pl.ds(48, 32)], acc_n.at[ixb], ssem1, add=True)

    def wait_scat(b):
        pkb, ixa, ixb, mpb, lsem = bufs[b]
        pltpu.make_async_copy(mpb.at[pl.ds(0, 48)], acc_p.at[ixa], ssem0).wait()
        pltpu.make_async_copy(mnb.at[pl.ds(0, 48)], acc_n.at[ixa], ssem0).wait()
        pltpu.make_async_copy(mpb.at[pl.ds(48, 32)], acc_p.at[ixb], ssem1).wait()
        pltpu.make_async_copy(mnb.at[pl.ds(48, 32)], acc_n.at[ixb], ssem1).wait()

    start_load(0, 0)

    zero16 = jnp.zeros((16,), jnp.float32)

    def _zrow(i, carry):
        for j in range(8):
            mnb[i, pl.ds(16 * j, 16)] = zero16
        return carry

    lax.fori_loop(0, CH, _zrow, 0)
    for kc in range(4):
        pltpu.sync_copy(mnb, acc_p.at[pl.ds(sid * 320 + kc * CH, CH)])
        pltpu.sync_copy(mnb, acc_n.at[pl.ds(sid * 320 + kc * CH, CH)])

    plsc.subcore_barrier()

    pltpu.sync_copy(s_hbm, wloc)

    def _maxb(g, m):
        return jnp.maximum(m, wloc[pl.ds(16 * g, 16)])

    mv = lax.fori_loop(1, NPAD // 16, _maxb, wloc[pl.ds(0, 16)])
    red16[pl.ds(0, 16)] = mv
    gmax = plsc.load_gather(red16, [jnp.zeros((16,), jnp.int32)])
    for j in range(1, 16):
        gmax = jnp.maximum(
            gmax, plsc.load_gather(red16, [jnp.full((16,), j, jnp.int32)]))

    def _expb(g, a):
        ev = jnp.exp(wloc[pl.ds(16 * g, 16)] - gmax)
        wloc[pl.ds(16 * g, 16)] = ev
        return a + ev

    zacc = lax.fori_loop(0, NPAD // 16, _expb, jnp.zeros((16,), jnp.float32))
    red16[pl.ds(0, 16)] = zacc
    zsum = plsc.load_gather(red16, [jnp.zeros((16,), jnp.int32)])
    for j in range(1, 16):
        zsum = zsum + plsc.load_gather(red16, [jnp.full((16,), j, jnp.int32)])
    inv_z = 1.0 / zsum

    def compute_part(b, g0, g1, ix):
        pkb, ixa, ixb, mpb, lsem = bufs[b]

        def _group(g, gc):
            pk = pkb[pl.ds(16 * g, 16)]
            sv = pk & 0xFFFF
            dv = lax.shift_right_logical(pk, 16)
            owned = (dv >= base_node) & (dv < base_node + NHALF)
            ix[pl.ds(16 * (g - g0), 16)] = jnp.where(owned, dv - base_node,
                                                     DUMP_ROW)
            wv = plsc.load_gather(wloc, [sv]) * inv_z
            for je in range(16):
                sp = wv[je]
                row = 16 * g + je
                for j in range(8):
                    rv = mpb[row, pl.ds(16 * j, 16)]
                    pv = jnp.maximum(rv, 0.0)
                    mpb[row, pl.ds(16 * j, 16)] = pv * sp
                    mnb[row, pl.ds(16 * j, 16)] = (pv - rv) * sp
            return gc

        lax.fori_loop(g0, g1, _group, 0)

    def step(ci, b):
        @pl.when(ci >= 1)
        def _():
            wait_scat(1 - b)

        @pl.when(ci < NCH - 1)
        def _():
            start_load(ci + 1, 1 - b)

        wait_load(b)
        pkb, ixa, ixb, mpb, lsem = bufs[b]
        compute_part(b, 0, 3, ixa)
        start_scat_a(b)
        compute_part(b, 3, 5, ixb)
        start_scat_b(b)

    def _loop(ci, carry):
        @pl.when(ci % 2 == 0)
        def _():
            step(ci, 0)

        @pl.when(ci % 2 == 1)
        def _():
            step(ci, 1)

        return carry

    lax.fori_loop(0, NCH, _loop, 0)
    wait_scat((NCH - 1) % 2)
    plsc.subcore_barrier()

    r0 = sid * 320
    pltpu.sync_copy(acc_p.at[pl.ds(r0, 320)], p_out.at[c, pl.ds(r0, 320)])
    pltpu.sync_copy(acc_n.at[pl.ds(r0, 320)], n_out.at[c, pl.ds(r0, 320)])


_sc_step = functools.partial(
    pl.kernel,
    out_type=(
        jax.ShapeDtypeStruct((2, ACC_ROWS, H), jnp.float32),
        jax.ShapeDtypeStruct((2, ACC_ROWS, H), jnp.float32),
    ),
    mesh=plsc.VectorSubcoreMesh(core_axis_name="c", subcore_axis_name="s"),
    compiler_params=pltpu.CompilerParams(needs_layout_passes=False),
    scratch_types=[
        pltpu.VMEM((NPAD,), jnp.float32),
        pltpu.VMEM((128,), jnp.float32),
        pltpu.VMEM((CH,), jnp.int32),
        pltpu.VMEM((48,), jnp.int32),
        pltpu.VMEM((32,), jnp.int32),
        pltpu.VMEM((CH, H), jnp.float32),
        pltpu.VMEM((CH,), jnp.int32),
        pltpu.VMEM((48,), jnp.int32),
        pltpu.VMEM((32,), jnp.int32),
        pltpu.VMEM((CH, H), jnp.float32),
        pltpu.VMEM((CH, H), jnp.float32),
        pltpu.VMEM_SHARED((ACC_ROWS, H), jnp.float32),
        pltpu.VMEM_SHARED((ACC_ROWS, H), jnp.float32),
        pltpu.SemaphoreType.DMA,
        pltpu.SemaphoreType.DMA,
        pltpu.SemaphoreType.DMA,
        pltpu.SemaphoreType.DMA,
    ],
)(_sc_body)


def _dense_body(h_ref, h0_ref, p_ref, ng_ref, wht_ref, mp_ref, mn_ref,
                bf1_ref, wf2t_ref, bf2_ref, wiht_ref, bih_ref, whht_ref,
                bhh_ref, lng_ref, lnb_ref, wot_ref, bo_ref, hn_ref, s_ref):
    f32 = jnp.float32
    hb = h_ref[...]
    u = jnp.dot(hb, wht_ref[...], preferred_element_type=f32)
    u = u + jnp.dot(p_ref[0], mp_ref[...], preferred_element_type=f32)
    u = u + jnp.dot(ng_ref[0], mn_ref[...], preferred_element_type=f32)
    u = u + bf1_ref[...]
    g = u * 0.5 * (1.0 + lax.erf(u * 0.7071067811865476))
    a = jnp.dot(g, wf2t_ref[...], preferred_element_type=f32)
    a = a + bf2_ref[...] + 0.1 * h0_ref[...]
    gi = jnp.dot(a, wiht_ref[...], preferred_element_type=f32) + bih_ref[...]
    gh = jnp.dot(hb, whht_ref[...], preferred_element_type=f32) + bhh_ref[...]
    r = jax.nn.sigmoid(gi[:, 0:H] + gh[:, 0:H])
    z = jax.nn.sigmoid(gi[:, H:2 * H] + gh[:, H:2 * H])
    ng = jnp.tanh(gi[:, 2 * H:3 * H] + r * gh[:, 2 * H:3 * H])
    hn = (1.0 - z) * ng + z * hb
    mu = jnp.mean(hn, axis=-1, keepdims=True)
    var = jnp.mean((hn - mu) ** 2, axis=-1, keepdims=True)
    hn = (hn - mu) * lax.rsqrt(var + 1e-5) * lng_ref[...] + lnb_ref[...]
    hn_ref[...] = hn
    s_ref[...] = jnp.dot(hn, wot_ref[...], preferred_element_type=f32) + bo_ref[...]


_BLK = 1000


def _dense_call(h, h0, p, ngv, wht, mp, mn, bf1, wf2t, bf2, wiht, bih,
                whht, bhh, lng, lnb, wot, bo):
    full = lambda shape: pl.BlockSpec(shape, lambda i: (0,) * len(shape))
    return pl.pallas_call(
        _dense_body,
        grid=(N // _BLK,),
        in_specs=[
            pl.BlockSpec((_BLK, H), lambda i: (i, 0)),
            pl.BlockSpec((_BLK, H), lambda i: (i, 0)),
            pl.BlockSpec((1, _BLK, H), lambda i: (i // 5, i % 5, 0)),
            pl.BlockSpec((1, _BLK, H), lambda i: (i // 5, i % 5, 0)),
            full((H, H)), full((H, H)), full((H, H)),
            full((1, H)), full((H, H)), full((1, H)),
            full((H, 3 * H)), full((1, 3 * H)), full((H, 3 * H)),
            full((1, 3 * H)), full((1, H)), full((1, H)),
            full((H, 1)), full((1, 1)),
        ],
        out_specs=[
            pl.BlockSpec((_BLK, H), lambda i: (i, 0)),
            pl.BlockSpec((_BLK, 1), lambda i: (i, 0)),
        ],
        out_shape=[
            jax.ShapeDtypeStruct((N, H), jnp.float32),
            jax.ShapeDtypeStruct((N, 1), jnp.float32),
        ],
    )(h, h0, p, ngv, wht, mp, mn, bf1, wf2t, bf2, wiht, bih, whht, bhh,
      lng, lnb, wot, bo)


def _matmul_bias_body(x_ref, w_ref, b_ref, o_ref):
    o_ref[...] = jnp.dot(x_ref[...], w_ref[...],
                         preferred_element_type=jnp.float32) + b_ref[...]


def _matmul_bias(x, w, b, blk):
    rows = x.shape[0]
    return pl.pallas_call(
        _matmul_bias_body,
        grid=(rows // blk,),
        in_specs=[
            pl.BlockSpec((blk, H), lambda i: (i, 0)),
            pl.BlockSpec((H, H), lambda i: (0, 0)),
            pl.BlockSpec((1, H), lambda i: (0, 0)),
        ],
        out_specs=pl.BlockSpec((blk, H), lambda i: (i, 0)),
        out_shape=jax.ShapeDtypeStruct((rows, H), jnp.float32),
    )(x, w, b)


def kernel(node_emb, node_mask, seed_mask, edge_src, edge_dst, edge_rel_emb,
           edge_dir, edge_mask, q_emb, Wn, bn, Wr, br, Wq, bq, Wins, bins,
           Wf1, bf1, Wf2, bf2, Wih, bih, Whh, bhh, ln_g, ln_b, Wsc, bsc,
           Wout, bout):
    f32 = jnp.float32
    src = edge_src[0].astype(jnp.int32)
    dst = edge_dst[0].astype(jnp.int32)
    pk = src | (dst << 16)

    qh = (q_emb @ Wq.T + bq)[0]
    ins = (qh @ Wins.T + bins).reshape(K, H)
    rk = jax.nn.relu(ins)
    sk = jax.nn.relu(-ins)
    mp = jnp.zeros((H, H), f32)
    mn = jnp.zeros((H, H), f32)
    for k in range(K):
        wk_t = Wf1[:, (1 + k) * H:(2 + k) * H].T
        mp = mp + rk[k][:, None] * wk_t
        mn = mn + sk[k][:, None] * wk_t
    wht = Wf1[:, :H].T

    h = _matmul_bias(node_emb[0], Wn.T, (bn + qh).reshape(1, H), _BLK)
    rel = _matmul_bias(edge_rel_emb[0], Wr.T, br.reshape(1, H), 2000)
    h0 = h

    pad = jnp.full((NPAD - N,), -1e30, f32)
    s = jnp.concatenate(
        [jnp.where(seed_mask[0], 0.0, -1e4).astype(f32), pad])

    wf2t = Wf2.T
    wiht = Wih.T
    whht = Whh.T
    bf1r = bf1.reshape(1, H)
    bf2r = bf2.reshape(1, H)
    bihr = bih.reshape(1, 3 * H)
    bhhr = bhh.reshape(1, 3 * H)
    lngr = ln_g.reshape(1, H)
    lnbr = ln_b.reshape(1, H)
    wsct = Wsc.T
    bscr = bsc.reshape(1, 1)
    woutt = Wout.T
    boutr = bout.reshape(1, 1)

    for step in range(R):
        p_acc, n_acc = _sc_step(s, rel, pk)
        last = step == R - 1
        h, s2d = _dense_call(
            h, h0, p_acc, n_acc, wht, mp, mn, bf1r, wf2t, bf2r, wiht, bihr,
            whht, bhhr, lngr, lnbr,
            woutt if last else wsct, boutr if last else bscr)
        if not last:
            s = jnp.concatenate([s2d[:, 0], pad])

    return s2d[:, 0][None, :]
